# baseline (speedup 1.0000x reference)
import functools

import numpy as np
import jax
import jax.numpy as jnp
from jax import lax
from jax.experimental import pallas as pl
from jax.experimental.pallas import tpu as pltpu
from jax.experimental.pallas import tpu_sc as plsc

_LANES = 16
_NW = 32
_B = 80

_GATHER_DN = lax.GatherDimensionNumbers(
    offset_dims=(), collapsed_slice_dims=(0,), start_index_map=(0,))


def _bcast_lane(vec, k):
    idx = jnp.full((_LANES, 1), k, jnp.int32)
    return lax.gather(vec, idx, _GATHER_DN, (1,),
                      mode=lax.GatherScatterMode.PROMISE_IN_BOUNDS)


def _transform(nodes, weights_t):
    r, hfr, hto = weights_t.shape[0], nodes.shape[1], weights_t.shape[2]
    n = nodes.shape[0]
    bn = 2000
    assert n % bn == 0 and hto % 32 == 0

    def body(x_ref, w_ref, o_ref):
        y = jnp.dot(x_ref[...], w_ref[0],
                    preferred_element_type=jnp.float32)
        b = pltpu.bitcast(y, jnp.int32)
        rnd = b + jnp.int32(0x7FFF) + ((b >> 16) & 1)
        lo = (rnd >> 16) & jnp.int32(0xFFFF)
        hi = rnd & jnp.int32(-65536)
        words = [lo[:, 32 * c:32 * c + 16] | hi[:, 32 * c + 16:32 * c + 32]
                 for c in range(hto // 32)]
        o_ref[0] = jnp.concatenate(words, axis=1)

    return pl.pallas_call(
        body,
        grid=(n // bn, r),
        in_specs=[
            pl.BlockSpec((bn, hfr), lambda bi, ri: (bi, 0)),
            pl.BlockSpec((1, hfr, hto), lambda bi, ri: (ri, 0, 0)),
        ],
        out_specs=pl.BlockSpec((1, bn, hto // 2), lambda bi, ri: (ri, bi, 0)),
        out_shape=jax.ShapeDtypeStruct((r, n, hto // 2), jnp.int32),
    )(nodes, weights_t)


def _make_sc_spmm(n, hto, nnz):
    assert nnz % _NW == 0
    htw = hto // 2
    ept = nnz // _NW
    assert ept % _B == 0 and ept % _LANES == 0
    nblk = ept // _B
    npair = (nblk - 1) // 2
    assert nblk == 2 * npair + 1
    rpt = (n // _LANES) // 8 * 8
    assert 0 <= n - _LANES * rpt <= _B

    mesh = plsc.VectorSubcoreMesh(core_axis_name="c", subcore_axis_name="s")

    @functools.partial(
        pl.kernel,
        out_type=jax.ShapeDtypeStruct((2, n, hto), jnp.float32),
        mesh=mesh,
        compiler_params=pltpu.CompilerParams(
            needs_layout_passes=False, use_tc_tiling_on_sc=False),
        scratch_types=[
            pltpu.VMEM((ept,), jnp.float32),
            pltpu.VMEM((ept,), jnp.int32),
            pltpu.VMEM((ept,), jnp.int32),
            pltpu.VMEM((_B, htw), jnp.int32),
            pltpu.VMEM((_B, htw), jnp.int32),
            pltpu.VMEM((_B, hto), jnp.float32),
            pltpu.VMEM_SHARED((n, hto), jnp.float32),
            pltpu.SemaphoreType.DMA,
            pltpu.SemaphoreType.DMA,
        ],
    )
    def sc_spmm(t_hbm, rows_hbm, cols_hbm, vals_hbm, out_hbm,
                vv, gv, dv, buf0, buf1, stg, accum, sem0, sem1):
        cid = lax.axis_index("c")
        sid = lax.axis_index("s")
        wid = cid * _LANES + sid

        zero16 = jnp.zeros((_LANES,), jnp.float32)
        for e in range(_B):
            for c8 in range(hto // _LANES):
                stg[e, pl.ds(c8 * _LANES, _LANES)] = zero16
        zbase = sid * rpt
        nfull = rpt // _B
        for k in range(nfull):
            pltpu.sync_copy(stg, accum.at[pl.ds(zbase + k * _B, _B)])
        rem = rpt - nfull * _B
        if rem:
            pltpu.sync_copy(stg.at[pl.ds(0, rem)],
                            accum.at[pl.ds(zbase + nfull * _B, rem)])
        tail = n - _LANES * rpt

        @pl.when(sid == _LANES - 1)
        def _zero_tail():
            pltpu.sync_copy(stg.at[pl.ds(0, tail)],
                            accum.at[pl.ds(_LANES * rpt, tail)])

        ebase = wid * ept
        pltpu.sync_copy(rows_hbm.at[pl.ds(ebase, ept)], gv)
        pltpu.sync_copy(cols_hbm.at[pl.ds(ebase, ept)], dv)
        pltpu.sync_copy(vals_hbm.at[pl.ds(ebase, ept)], vv)

        plsc.subcore_barrier()

        n_vec = jnp.full((_LANES,), n, jnp.int32)

        def idx_body(i, carry):
            sl = pl.ds(i * _LANES, _LANES)
            r16 = gv[sl]
            c16 = dv[sl]
            d16 = lax.rem(r16, n_vec)
            gv[sl] = (r16 - d16) + c16
            dv[sl] = d16
            return carry

        lax.fori_loop(0, ept // _LANES, idx_body, 0)

        hi_mask = jnp.full((_LANES,), -65536, jnp.int32)

        def scale(buf, vbase):
            for j in range(_B // _LANES):
                v16 = vv[pl.ds(vbase + j * _LANES, _LANES)]
                for k in range(_LANES):
                    ve = _bcast_lane(v16, k)
                    e = j * _LANES + k
                    for c in range(hto // 32):
                        w16 = buf[e, pl.ds(c * _LANES, _LANES)]
                        a = plsc.bitcast(w16 << 16, jnp.float32)
                        b = plsc.bitcast(w16 & hi_mask, jnp.float32)
                        sl = pl.ds(2 * c * _LANES, _LANES)
                        sh = pl.ds((2 * c + 1) * _LANES, _LANES)
                        stg[e, sl] = a * ve
                        stg[e, sh] = b * ve

        def gather(blk, buf, sem):
            pltpu.async_copy(
                t_hbm.at[gv.at[pl.ds(blk * _B, _B)]], buf, sem)

        def gather_wait(blk, buf, sem):
            pltpu.make_async_copy(
                t_hbm.at[gv.at[pl.ds(blk * _B, _B)]], buf, sem).wait()

        def scatter_add(blk):
            pltpu.sync_copy(stg, accum.at[dv.at[pl.ds(blk * _B, _B)]],
                            add=True)

        gather(0, buf0, sem0)

        def body(i, carry):
            p0 = 2 * i
            p1 = p0 + 1
            gather(p1, buf1, sem1)
            gather_wait(p0, buf0, sem0)
            scale(buf0, p0 * _B)
            scatter_add(p0)

            @pl.when(p0 + 2 < nblk)
            def _prefetch():
                gather(p0 + 2, buf0, sem0)

            gather_wait(p1, buf1, sem1)
            scale(buf1, p1 * _B)
            scatter_add(p1)
            return carry

        lax.fori_loop(0, npair, body, 0)

        gather_wait(nblk - 1, buf0, sem0)
        scale(buf0, (nblk - 1) * _B)
        scatter_add(nblk - 1)

        plsc.subcore_barrier()
        obase = sid * rpt
        pltpu.sync_copy(accum.at[pl.ds(obase, rpt)],
                        out_hbm.at[cid, pl.ds(obase, rpt)])

        @pl.when(sid == _LANES - 1)
        def _copy_tail():
            pltpu.sync_copy(accum.at[pl.ds(_LANES * rpt, tail)],
                            out_hbm.at[cid, pl.ds(_LANES * rpt, tail)])

    return sc_spmm


def _finalize(partials):
    _, n, hto = partials.shape
    bn = 2000
    assert n % bn == 0

    def body(p_ref, o_ref):
        o_ref[...] = jnp.maximum(p_ref[0] + p_ref[1], 0.0)

    return pl.pallas_call(
        body,
        grid=(n // bn,),
        in_specs=[pl.BlockSpec((2, bn, hto), lambda i: (0, i, 0))],
        out_specs=pl.BlockSpec((bn, hto), lambda i: (i, 0)),
        out_shape=jax.ShapeDtypeStruct((n, hto), jnp.float32),
    )(partials)


def kernel(nodes, indices, values, weights):
    n, hfr = nodes.shape
    r, _, hto = weights.shape
    nnz = values.shape[0]

    weights_t = weights.transpose(0, 2, 1)
    t_table = _transform(nodes, weights_t).reshape(r * n, hto // 2)

    rows = indices[0].astype(jnp.int32)
    cols = indices[1].astype(jnp.int32)
    vals = values.astype(jnp.float32)

    partials = _make_sc_spmm(n, hto, nnz)(t_table, rows, cols, vals)
    return _finalize(partials)


# speedup vs baseline: 1.0776x; 1.0776x over previous
import functools

import numpy as np
import jax
import jax.numpy as jnp
from jax import lax
from jax.experimental import pallas as pl
from jax.experimental.pallas import tpu as pltpu
from jax.experimental.pallas import tpu_sc as plsc

_LANES = 16
_NW = 32
_B = 80

_GATHER_DN = lax.GatherDimensionNumbers(
    offset_dims=(), collapsed_slice_dims=(0,), start_index_map=(0,))


def _bcast_lane(vec, k):
    idx = jnp.full((_LANES, 1), k, jnp.int32)
    return lax.gather(vec, idx, _GATHER_DN, (1,),
                      mode=lax.GatherScatterMode.PROMISE_IN_BOUNDS)


def _transform(nodes_even, nodes_odd, weights_t):
    r, hfr, hto = weights_t.shape[0], nodes_even.shape[1], weights_t.shape[2]
    n2 = nodes_even.shape[0]
    bn = 1000
    assert n2 % bn == 0 and hto % 32 == 0

    def pack(y):
        b = pltpu.bitcast(y, jnp.int32)
        rnd = b + jnp.int32(0x7FFF) + ((b >> 16) & 1)
        lo = (rnd >> 16) & jnp.int32(0xFFFF)
        hi = rnd & jnp.int32(-65536)
        return [lo[:, 32 * c:32 * c + 16] | hi[:, 32 * c + 16:32 * c + 32]
                for c in range(hto // 32)]

    def body(xe_ref, xo_ref, w_ref, o_ref):
        ye = jnp.dot(xe_ref[...], w_ref[0],
                     preferred_element_type=jnp.float32)
        yo = jnp.dot(xo_ref[...], w_ref[0],
                     preferred_element_type=jnp.float32)
        o_ref[0] = jnp.concatenate(pack(ye) + pack(yo), axis=1)

    return pl.pallas_call(
        body,
        grid=(n2 // bn, r),
        in_specs=[
            pl.BlockSpec((bn, hfr), lambda bi, ri: (bi, 0)),
            pl.BlockSpec((bn, hfr), lambda bi, ri: (bi, 0)),
            pl.BlockSpec((1, hfr, hto), lambda bi, ri: (ri, 0, 0)),
        ],
        out_specs=pl.BlockSpec((1, bn, hto), lambda bi, ri: (ri, bi, 0)),
        out_shape=jax.ShapeDtypeStruct((r, n2, hto), jnp.int32),
    )(nodes_even, nodes_odd, weights_t)


def _make_sc_spmm(n, hto, nnz):
    assert nnz % _NW == 0
    htw = hto // 2
    ept = nnz // _NW
    assert ept % _B == 0 and ept % _LANES == 0
    nblk = ept // _B
    npair = (nblk - 1) // 2
    assert nblk == 2 * npair + 1
    rpt = (n // _LANES) // 8 * 8
    assert 0 <= n - _LANES * rpt <= _B

    mesh = plsc.VectorSubcoreMesh(core_axis_name="c", subcore_axis_name="s")

    @functools.partial(
        pl.kernel,
        out_type=jax.ShapeDtypeStruct((2, n, hto), jnp.float32),
        mesh=mesh,
        compiler_params=pltpu.CompilerParams(
            needs_layout_passes=False, use_tc_tiling_on_sc=False),
        scratch_types=[
            pltpu.VMEM((ept,), jnp.float32),
            pltpu.VMEM((ept,), jnp.int32),
            pltpu.VMEM((ept,), jnp.int32),
            pltpu.VMEM((_B, htw), jnp.int32),
            pltpu.VMEM((_B, htw), jnp.int32),
            pltpu.VMEM((_B, hto), jnp.float32),
            pltpu.VMEM_SHARED((n, hto), jnp.float32),
            pltpu.SemaphoreType.DMA,
            pltpu.SemaphoreType.DMA,
        ],
    )
    def sc_spmm(t_hbm, rows_hbm, cols_hbm, vals_hbm, out_hbm,
                vv, gv, dv, buf0, buf1, stg, accum, sem0, sem1):
        cid = lax.axis_index("c")
        sid = lax.axis_index("s")
        wid = cid * _LANES + sid

        zero16 = jnp.zeros((_LANES,), jnp.float32)
        for e in range(_B):
            for c8 in range(hto // _LANES):
                stg[e, pl.ds(c8 * _LANES, _LANES)] = zero16
        zbase = sid * rpt
        nfull = rpt // _B
        for k in range(nfull):
            pltpu.sync_copy(stg, accum.at[pl.ds(zbase + k * _B, _B)])
        rem = rpt - nfull * _B
        if rem:
            pltpu.sync_copy(stg.at[pl.ds(0, rem)],
                            accum.at[pl.ds(zbase + nfull * _B, rem)])
        tail = n - _LANES * rpt

        @pl.when(sid == _LANES - 1)
        def _zero_tail():
            pltpu.sync_copy(stg.at[pl.ds(0, tail)],
                            accum.at[pl.ds(_LANES * rpt, tail)])

        ebase = wid * ept
        pltpu.sync_copy(rows_hbm.at[pl.ds(ebase, ept)], gv)
        pltpu.sync_copy(cols_hbm.at[pl.ds(ebase, ept)], dv)
        pltpu.sync_copy(vals_hbm.at[pl.ds(ebase, ept)], vv)

        plsc.subcore_barrier()

        n_vec = jnp.full((_LANES,), n, jnp.int32)

        def idx_body(i, carry):
            sl = pl.ds(i * _LANES, _LANES)
            r16 = gv[sl]
            c16 = dv[sl]
            d16 = lax.rem(r16, n_vec)
            gv[sl] = (r16 - d16) + c16
            dv[sl] = d16
            return carry

        lax.fori_loop(0, ept // _LANES, idx_body, 0)

        hi_mask = jnp.full((_LANES,), -65536, jnp.int32)

        def scale(buf, vbase):
            for j in range(_B // _LANES):
                v16 = vv[pl.ds(vbase + j * _LANES, _LANES)]
                for k in range(_LANES):
                    ve = _bcast_lane(v16, k)
                    e = j * _LANES + k
                    for c in range(hto // 32):
                        w16 = buf[e, pl.ds(c * _LANES, _LANES)]
                        a = plsc.bitcast(w16 << 16, jnp.float32)
                        b = plsc.bitcast(w16 & hi_mask, jnp.float32)
                        sl = pl.ds(2 * c * _LANES, _LANES)
                        sh = pl.ds((2 * c + 1) * _LANES, _LANES)
                        stg[e, sl] = a * ve
                        stg[e, sh] = b * ve

        def gather(blk, buf, sem):
            pltpu.async_copy(
                t_hbm.at[gv.at[pl.ds(blk * _B, _B)]], buf, sem)

        def gather_wait(blk, buf, sem):
            pltpu.make_async_copy(
                t_hbm.at[gv.at[pl.ds(blk * _B, _B)]], buf, sem).wait()

        def scatter_add(blk):
            pltpu.sync_copy(stg, accum.at[dv.at[pl.ds(blk * _B, _B)]],
                            add=True)

        gather(0, buf0, sem0)

        def body(i, carry):
            p0 = 2 * i
            p1 = p0 + 1
            gather(p1, buf1, sem1)
            gather_wait(p0, buf0, sem0)
            scale(buf0, p0 * _B)
            scatter_add(p0)

            @pl.when(p0 + 2 < nblk)
            def _prefetch():
                gather(p0 + 2, buf0, sem0)

            gather_wait(p1, buf1, sem1)
            scale(buf1, p1 * _B)
            scatter_add(p1)
            return carry

        lax.fori_loop(0, npair, body, 0)

        gather_wait(nblk - 1, buf0, sem0)
        scale(buf0, (nblk - 1) * _B)
        scatter_add(nblk - 1)

        plsc.subcore_barrier()
        obase = sid * rpt
        pltpu.sync_copy(accum.at[pl.ds(obase, rpt)],
                        out_hbm.at[cid, pl.ds(obase, rpt)])

        @pl.when(sid == _LANES - 1)
        def _copy_tail():
            pltpu.sync_copy(accum.at[pl.ds(_LANES * rpt, tail)],
                            out_hbm.at[cid, pl.ds(_LANES * rpt, tail)])

    return sc_spmm


def _finalize(partials):
    _, n, hto = partials.shape
    bn = 2000
    assert n % bn == 0

    def body(p_ref, o_ref):
        o_ref[...] = jnp.maximum(p_ref[0] + p_ref[1], 0.0)

    return pl.pallas_call(
        body,
        grid=(n // bn,),
        in_specs=[pl.BlockSpec((2, bn, hto), lambda i: (0, i, 0))],
        out_specs=pl.BlockSpec((bn, hto), lambda i: (i, 0)),
        out_shape=jax.ShapeDtypeStruct((n, hto), jnp.float32),
    )(partials)


def kernel(nodes, indices, values, weights):
    n, hfr = nodes.shape
    r, _, hto = weights.shape
    nnz = values.shape[0]

    weights_t = weights.transpose(0, 2, 1)
    t_table = _transform(nodes[0::2], nodes[1::2],
                         weights_t).reshape(r * n, hto // 2)

    rows = indices[0].astype(jnp.int32)
    cols = indices[1].astype(jnp.int32)
    vals = values.astype(jnp.float32)

    partials = _make_sc_spmm(n, hto, nnz)(t_table, rows, cols, vals)
    return _finalize(partials)


# speedup vs baseline: 1.2394x; 1.1502x over previous
import functools

import numpy as np
import jax
import jax.numpy as jnp
from jax import lax
from jax.experimental import pallas as pl
from jax.experimental.pallas import tpu as pltpu
from jax.experimental.pallas import tpu_sc as plsc

_LANES = 16
_NW = 32
_B = 80

_GATHER_DN = lax.GatherDimensionNumbers(
    offset_dims=(), collapsed_slice_dims=(0,), start_index_map=(0,))


def _bcast_lane(vec, k):
    idx = jnp.full((_LANES, 1), k, jnp.int32)
    return lax.gather(vec, idx, _GATHER_DN, (1,),
                      mode=lax.GatherScatterMode.PROMISE_IN_BOUNDS)


def _transform(nodes_even, nodes_odd, weights_t):
    r, hfr, hto = weights_t.shape[0], nodes_even.shape[1], weights_t.shape[2]
    n2 = nodes_even.shape[0]
    bn = 1000
    assert n2 % bn == 0 and hto % 32 == 0

    def pack(y):
        b = pltpu.bitcast(y, jnp.int32)
        rnd = b + jnp.int32(0x7FFF) + ((b >> 16) & 1)
        lo = (rnd[:, :hto // 2] >> 16) & jnp.int32(0xFFFF)
        hi = rnd[:, hto // 2:] & jnp.int32(-65536)
        return lo | hi

    def body(xe_ref, xo_ref, w_ref, o_ref):
        ye = jnp.dot(xe_ref[...], w_ref[0],
                     preferred_element_type=jnp.float32)
        yo = jnp.dot(xo_ref[...], w_ref[0],
                     preferred_element_type=jnp.float32)
        o_ref[0] = jnp.concatenate([pack(ye), pack(yo)], axis=1)

    return pl.pallas_call(
        body,
        grid=(n2 // bn, r),
        in_specs=[
            pl.BlockSpec((bn, hfr), lambda bi, ri: (bi, 0)),
            pl.BlockSpec((bn, hfr), lambda bi, ri: (bi, 0)),
            pl.BlockSpec((1, hfr, hto), lambda bi, ri: (ri, 0, 0)),
        ],
        out_specs=pl.BlockSpec((1, bn, hto), lambda bi, ri: (ri, bi, 0)),
        out_shape=jax.ShapeDtypeStruct((r, n2, hto), jnp.int32),
    )(nodes_even, nodes_odd, weights_t)


def _make_sc_spmm(n, hto, nnz):
    assert nnz % _NW == 0
    htw = hto // 2
    ept = nnz // _NW
    assert ept % _B == 0 and ept % _LANES == 0
    nblk = ept // _B
    npair = (nblk - 1) // 2
    assert nblk == 2 * npair + 1
    rpt = (n // _LANES) // 8 * 8
    assert 0 <= n - _LANES * rpt <= _B

    mesh = plsc.VectorSubcoreMesh(core_axis_name="c", subcore_axis_name="s")

    @functools.partial(
        pl.kernel,
        out_type=jax.ShapeDtypeStruct((2, n, hto), jnp.float32),
        mesh=mesh,
        compiler_params=pltpu.CompilerParams(
            needs_layout_passes=False, use_tc_tiling_on_sc=False),
        scratch_types=[
            pltpu.VMEM((ept,), jnp.float32),
            pltpu.VMEM((ept,), jnp.int32),
            pltpu.VMEM((ept,), jnp.int32),
            pltpu.VMEM((_B, htw), jnp.int32),
            pltpu.VMEM((_B, htw), jnp.int32),
            pltpu.VMEM((_B, hto), jnp.float32),
            pltpu.VMEM_SHARED((n, hto), jnp.float32),
            pltpu.SemaphoreType.DMA,
            pltpu.SemaphoreType.DMA,
        ],
    )
    def sc_spmm(t_hbm, rows_hbm, cols_hbm, vals_hbm, out_hbm,
                vv, gv, dv, buf0, buf1, stg, accum, sem0, sem1):
        cid = lax.axis_index("c")
        sid = lax.axis_index("s")
        wid = cid * _LANES + sid

        zero16 = jnp.zeros((_LANES,), jnp.float32)
        for e in range(_B):
            for c8 in range(hto // _LANES):
                stg[e, pl.ds(c8 * _LANES, _LANES)] = zero16
        zbase = sid * rpt
        nfull = rpt // _B
        for k in range(nfull):
            pltpu.sync_copy(stg, accum.at[pl.ds(zbase + k * _B, _B)])
        rem = rpt - nfull * _B
        if rem:
            pltpu.sync_copy(stg.at[pl.ds(0, rem)],
                            accum.at[pl.ds(zbase + nfull * _B, rem)])
        tail = n - _LANES * rpt

        @pl.when(sid == _LANES - 1)
        def _zero_tail():
            pltpu.sync_copy(stg.at[pl.ds(0, tail)],
                            accum.at[pl.ds(_LANES * rpt, tail)])

        ebase = wid * ept
        pltpu.sync_copy(rows_hbm.at[pl.ds(ebase, ept)], gv)
        pltpu.sync_copy(cols_hbm.at[pl.ds(ebase, ept)], dv)
        pltpu.sync_copy(vals_hbm.at[pl.ds(ebase, ept)], vv)

        plsc.subcore_barrier()

        n_vec = jnp.full((_LANES,), n, jnp.int32)

        def idx_body(i, carry):
            sl = pl.ds(i * _LANES, _LANES)
            r16 = gv[sl]
            c16 = dv[sl]
            d16 = lax.rem(r16, n_vec)
            gv[sl] = (r16 - d16) + c16
            dv[sl] = d16
            return carry

        lax.fori_loop(0, ept // _LANES, idx_body, 0)

        hi_mask = jnp.full((_LANES,), -65536, jnp.int32)

        def scale(buf, vbase):
            for j in range(_B // _LANES):
                v16 = vv[pl.ds(vbase + j * _LANES, _LANES)]
                for k in range(_LANES):
                    ve = _bcast_lane(v16, k)
                    e = j * _LANES + k
                    for c in range(hto // 32):
                        w16 = buf[e, pl.ds(c * _LANES, _LANES)]
                        a = plsc.bitcast(w16 << 16, jnp.float32)
                        b = plsc.bitcast(w16 & hi_mask, jnp.float32)
                        sl = pl.ds(c * _LANES, _LANES)
                        sh = pl.ds(hto // 2 + c * _LANES, _LANES)
                        stg[e, sl] = a * ve
                        stg[e, sh] = b * ve

        def gather(blk, buf, sem):
            pltpu.async_copy(
                t_hbm.at[gv.at[pl.ds(blk * _B, _B)]], buf, sem)

        def gather_wait(blk, buf, sem):
            pltpu.make_async_copy(
                t_hbm.at[gv.at[pl.ds(blk * _B, _B)]], buf, sem).wait()

        def scatter_add(blk):
            pltpu.sync_copy(stg, accum.at[dv.at[pl.ds(blk * _B, _B)]],
                            add=True)

        gather(0, buf0, sem0)

        def body(i, carry):
            p0 = 2 * i
            p1 = p0 + 1
            gather(p1, buf1, sem1)
            gather_wait(p0, buf0, sem0)
            scale(buf0, p0 * _B)
            scatter_add(p0)

            @pl.when(p0 + 2 < nblk)
            def _prefetch():
                gather(p0 + 2, buf0, sem0)

            gather_wait(p1, buf1, sem1)
            scale(buf1, p1 * _B)
            scatter_add(p1)
            return carry

        lax.fori_loop(0, npair, body, 0)

        gather_wait(nblk - 1, buf0, sem0)
        scale(buf0, (nblk - 1) * _B)
        scatter_add(nblk - 1)

        plsc.subcore_barrier()
        obase = sid * rpt
        pltpu.sync_copy(accum.at[pl.ds(obase, rpt)],
                        out_hbm.at[cid, pl.ds(obase, rpt)])

        @pl.when(sid == _LANES - 1)
        def _copy_tail():
            pltpu.sync_copy(accum.at[pl.ds(_LANES * rpt, tail)],
                            out_hbm.at[cid, pl.ds(_LANES * rpt, tail)])

    return sc_spmm


def _finalize(partials):
    _, n, hto = partials.shape
    bn = 2000
    assert n % bn == 0

    def body(p_ref, o_ref):
        o_ref[...] = jnp.maximum(p_ref[0] + p_ref[1], 0.0)

    return pl.pallas_call(
        body,
        grid=(n // bn,),
        in_specs=[pl.BlockSpec((2, bn, hto), lambda i: (0, i, 0))],
        out_specs=pl.BlockSpec((bn, hto), lambda i: (i, 0)),
        out_shape=jax.ShapeDtypeStruct((n, hto), jnp.float32),
    )(partials)


def kernel(nodes, indices, values, weights):
    n, hfr = nodes.shape
    r, _, hto = weights.shape
    nnz = values.shape[0]

    weights_t = weights.transpose(0, 2, 1)
    t_table = _transform(nodes[0::2], nodes[1::2],
                         weights_t).reshape(r * n, hto // 2)

    rows = indices[0].astype(jnp.int32)
    cols = indices[1].astype(jnp.int32)
    vals = values.astype(jnp.float32)

    partials = _make_sc_spmm(n, hto, nnz)(t_table, rows, cols, vals)
    return _finalize(partials)
